# baseline (speedup 1.0000x reference)
import jax
import jax.numpy as jnp
from jax.experimental import pallas as pl
from jax.experimental.pallas import tpu as pltpu

EPS_ = 1e-05
CH_ = 256
NCH_ = 32
TL_ = CH_ * NCH_
G_ = 4


def _revin_kernel(x_ref, u_ref, o_ref, c1_ref, c2_ref):
    tb = pl.program_id(1)

    @pl.when(tb == 0)
    def _():
        c1_ref[...] = jnp.zeros_like(c1_ref)
        c2_ref[...] = jnp.zeros_like(c2_ref)

    umat16 = u_ref[...]
    t0 = tb * TL_

    carry1 = [c1_ref[:, g : g + 1] for g in range(G_)]
    carry2 = [c2_ref[:, g : g + 1] for g in range(G_)]
    for c in range(NCH_):
        sl = slice(c * CH_, (c + 1) * CH_)
        nm1 = jax.lax.broadcasted_iota(jnp.int32, (1, CH_), 1) + (t0 + c * CH_)
        nm1f = nm1.astype(jnp.float32)
        inv_n = 1.0 / (nm1f + 1.0)
        for g in range(G_):
            zc = x_ref[g][:, sl]
            s1ex = (
                jax.lax.dot(
                    zc.astype(jnp.bfloat16), umat16,
                    preferred_element_type=jnp.float32,
                )
                + carry1[g]
            )
            carry1[g] = s1ex[:, CH_ - 1 :] + zc[:, CH_ - 1 :]

            d = (zc * nm1f - s1ex) * inv_n
            d2 = d * d
            s2ex = (
                jax.lax.dot(
                    d2.astype(jnp.bfloat16), umat16,
                    preferred_element_type=jnp.float32,
                )
                + carry2[g]
            )
            carry2[g] = s2ex[:, CH_ - 1 :] + d2[:, CH_ - 1 :]

            o_ref[g, :, sl] = d * jax.lax.rsqrt(
                jnp.maximum((s2ex + d2) * inv_n, EPS_)
            )

    for g in range(G_):
        c1_ref[:, g : g + 1] = carry1[g]
        c2_ref[:, g : g + 1] = carry2[g]


def kernel(x):
    b, t, c = x.shape
    xt = x.transpose(0, 2, 1)
    umat16 = jnp.triu(jnp.ones((CH_, CH_), jnp.bfloat16), k=1)

    out = pl.pallas_call(
        _revin_kernel,
        grid=(b // G_, t // TL_),
        in_specs=[
            pl.BlockSpec((G_, c, TL_), lambda i, j: (i, 0, j)),
            pl.BlockSpec((CH_, CH_), lambda i, j: (0, 0)),
        ],
        out_specs=pl.BlockSpec((G_, c, TL_), lambda i, j: (i, 0, j)),
        out_shape=jax.ShapeDtypeStruct(xt.shape, x.dtype),
        scratch_shapes=[
            pltpu.VMEM((c, G_), jnp.float32),
            pltpu.VMEM((c, G_), jnp.float32),
        ],
        compiler_params=pltpu.CompilerParams(
            dimension_semantics=("parallel", "arbitrary"),
        ),
    )(xt, umat16)
    return out.transpose(0, 2, 1)


# speedup vs baseline: 1.0591x; 1.0591x over previous
import jax
import jax.numpy as jnp
from jax.experimental import pallas as pl
from jax.experimental.pallas import tpu as pltpu

EPS_ = 1e-05
CH_ = 256
NCH_ = 16
TL_ = CH_ * NCH_
G_ = 8


def _revin_kernel(x_ref, u_ref, o_ref, c1_ref, c2_ref):
    tb = pl.program_id(1)

    @pl.when(tb == 0)
    def _():
        c1_ref[...] = jnp.zeros_like(c1_ref)
        c2_ref[...] = jnp.zeros_like(c2_ref)

    umat16 = u_ref[...]
    t0 = tb * TL_

    carry1 = [c1_ref[:, g : g + 1] for g in range(G_)]
    carry2 = [c2_ref[:, g : g + 1] for g in range(G_)]
    for c in range(NCH_):
        sl = slice(c * CH_, (c + 1) * CH_)
        nm1 = jax.lax.broadcasted_iota(jnp.int32, (1, CH_), 1) + (t0 + c * CH_)
        nm1f = nm1.astype(jnp.float32)
        inv_n = 1.0 / (nm1f + 1.0)
        for g in range(G_):
            zc = x_ref[g][:, sl]
            s1ex = (
                jax.lax.dot(
                    zc.astype(jnp.bfloat16), umat16,
                    preferred_element_type=jnp.float32,
                )
                + carry1[g]
            )
            carry1[g] = s1ex[:, CH_ - 1 :] + zc[:, CH_ - 1 :]

            d = (zc * nm1f - s1ex) * inv_n
            d2 = d * d
            s2ex = (
                jax.lax.dot(
                    d2.astype(jnp.bfloat16), umat16,
                    preferred_element_type=jnp.float32,
                )
                + carry2[g]
            )
            carry2[g] = s2ex[:, CH_ - 1 :] + d2[:, CH_ - 1 :]

            o_ref[g, :, sl] = d * jax.lax.rsqrt(
                jnp.maximum((s2ex + d2) * inv_n, EPS_)
            )

    for g in range(G_):
        c1_ref[:, g : g + 1] = carry1[g]
        c2_ref[:, g : g + 1] = carry2[g]


def kernel(x):
    b, t, c = x.shape
    xt = x.transpose(0, 2, 1)
    umat16 = jnp.triu(jnp.ones((CH_, CH_), jnp.bfloat16), k=1)

    out = pl.pallas_call(
        _revin_kernel,
        grid=(b // G_, t // TL_),
        in_specs=[
            pl.BlockSpec((G_, c, TL_), lambda i, j: (i, 0, j)),
            pl.BlockSpec((CH_, CH_), lambda i, j: (0, 0)),
        ],
        out_specs=pl.BlockSpec((G_, c, TL_), lambda i, j: (i, 0, j)),
        out_shape=jax.ShapeDtypeStruct(xt.shape, x.dtype),
        scratch_shapes=[
            pltpu.VMEM((c, G_), jnp.float32),
            pltpu.VMEM((c, G_), jnp.float32),
        ],
        compiler_params=pltpu.CompilerParams(
            dimension_semantics=("parallel", "arbitrary"),
        ),
    )(xt, umat16)
    return out.transpose(0, 2, 1)
